# baseline (speedup 1.0000x reference)
import functools

import jax
import jax.numpy as jnp
from jax import lax
from jax.experimental import pallas as pl
from jax.experimental.pallas import tpu as pltpu

_INV_SQRT2 = 0.7071067811865476


def _adapter_body(cid_ref, h_ref, wd_ref, bd_ref, wu_ref, bu_ref, out_ref,
                  wd_s, wu_s, bd_s, *, bd_dim):
    n_e = wd_ref.shape[0]

    @pl.when(pl.program_id(0) == 0)
    def _build_weights():
        for e in range(n_e):
            sl = pl.ds(e * bd_dim, bd_dim)
            wd_s[:, sl] = wd_ref[e].astype(jnp.bfloat16)
            wu_s[sl, :] = wu_ref[e].astype(jnp.bfloat16)
            bd_s[0:1, sl] = bd_ref[pl.ds(e, 1), :]

    hb = h_ref[...]
    cid = cid_ref[...]
    h16 = hb.astype(jnp.bfloat16)
    z = jnp.dot(h16, wd_s[...], preferred_element_type=jnp.float32)
    z = z + bd_s[...]
    a = 0.5 * z * (1.0 + lax.erf(z * _INV_SQRT2))
    col_expert = lax.broadcasted_iota(jnp.int32, z.shape, 1) // bd_dim
    am = jnp.where(col_expert == cid, a, 0.0).astype(jnp.bfloat16)
    delta = jnp.dot(am, wu_s[...], preferred_element_type=jnp.float32)
    oh = (lax.broadcasted_iota(jnp.int32, (hb.shape[0], n_e), 1) == cid)
    bu_sel = jnp.dot(oh.astype(jnp.float32), bu_ref[...],
                     preferred_element_type=jnp.float32)
    out_ref[...] = hb + delta + bu_sel


def kernel(h, cluster_ids, W_down, b_down, W_up, b_up):
    B, D = h.shape
    E, _, BD = W_down.shape
    BT = 256

    cid2 = cluster_ids.astype(jnp.int32).reshape(B, 1)

    return pl.pallas_call(
        functools.partial(_adapter_body, bd_dim=BD),
        grid=(B // BT,),
        in_specs=[
            pl.BlockSpec((BT, 1), lambda i: (i, 0)),
            pl.BlockSpec((BT, D), lambda i: (i, 0)),
            pl.BlockSpec((E, D, BD), lambda i: (0, 0, 0)),
            pl.BlockSpec((E, BD), lambda i: (0, 0)),
            pl.BlockSpec((E, BD, D), lambda i: (0, 0, 0)),
            pl.BlockSpec((E, D), lambda i: (0, 0)),
        ],
        out_specs=pl.BlockSpec((BT, D), lambda i: (i, 0)),
        out_shape=jax.ShapeDtypeStruct((B, D), jnp.float32),
        scratch_shapes=[
            pltpu.VMEM((D, E * BD), jnp.bfloat16),
            pltpu.VMEM((E * BD, D), jnp.bfloat16),
            pltpu.VMEM((1, E * BD), jnp.float32),
        ],
    )(cid2, h, W_down, b_down, W_up, b_up)


# speedup vs baseline: 1.1125x; 1.1125x over previous
import functools

import jax
import jax.numpy as jnp
from jax import lax
from jax.experimental import pallas as pl
from jax.experimental.pallas import tpu as pltpu

_INV_SQRT2 = 0.7071067811865476


def _adapter_body(cid_ref, h_ref, wd_ref, bd_ref, wu_ref, bu_ref, out_ref,
                  wd_s, wu_s, bd_s, *, bd_dim):
    n_e = wd_ref.shape[0]

    @pl.when(pl.program_id(0) == 0)
    def _build_weights():
        for e in range(n_e):
            sl = pl.ds(e * bd_dim, bd_dim)
            wd_s[:, sl] = wd_ref[e].astype(jnp.bfloat16)
            wu_s[sl, :] = wu_ref[e].astype(jnp.bfloat16)
            bd_s[0:1, sl] = bd_ref[pl.ds(e, 1), :]

    hb = h_ref[...]
    cid = cid_ref[...]
    h16 = hb.astype(jnp.bfloat16)
    z = jnp.dot(h16, wd_s[...], preferred_element_type=jnp.float32)
    z = z + bd_s[...]
    a = 0.5 * z * (1.0 + lax.erf(z * _INV_SQRT2))
    col_expert = lax.broadcasted_iota(jnp.int32, z.shape, 1) // bd_dim
    am = jnp.where(col_expert == cid, a, 0.0).astype(jnp.bfloat16)
    delta = jnp.dot(am, wu_s[...], preferred_element_type=jnp.float32)
    oh = (lax.broadcasted_iota(jnp.int32, (hb.shape[0], n_e), 1) == cid)
    bu_sel = jnp.dot(oh.astype(jnp.float32), bu_ref[...],
                     preferred_element_type=jnp.float32)
    out_ref[...] = hb + delta + bu_sel


def kernel(h, cluster_ids, W_down, b_down, W_up, b_up):
    B, D = h.shape
    E, _, BD = W_down.shape
    BT = 1024

    cid2 = cluster_ids.astype(jnp.int32).reshape(B, 1)

    return pl.pallas_call(
        functools.partial(_adapter_body, bd_dim=BD),
        grid=(B // BT,),
        in_specs=[
            pl.BlockSpec((BT, 1), lambda i: (i, 0)),
            pl.BlockSpec((BT, D), lambda i: (i, 0)),
            pl.BlockSpec((E, D, BD), lambda i: (0, 0, 0)),
            pl.BlockSpec((E, BD), lambda i: (0, 0)),
            pl.BlockSpec((E, BD, D), lambda i: (0, 0, 0)),
            pl.BlockSpec((E, D), lambda i: (0, 0)),
        ],
        out_specs=pl.BlockSpec((BT, D), lambda i: (i, 0)),
        out_shape=jax.ShapeDtypeStruct((B, D), jnp.float32),
        scratch_shapes=[
            pltpu.VMEM((D, E * BD), jnp.bfloat16),
            pltpu.VMEM((E * BD, D), jnp.bfloat16),
            pltpu.VMEM((1, E * BD), jnp.float32),
        ],
    )(cid2, h, W_down, b_down, W_up, b_up)


# speedup vs baseline: 1.2675x; 1.1393x over previous
import functools

import jax
import jax.numpy as jnp
from jax import lax
from jax.experimental import pallas as pl
from jax.experimental.pallas import tpu as pltpu

_INV_SQRT2 = 0.7071067811865476


def _adapter_body(cid_ref, h_ref, wd_ref, bd_ref, wu_ref, bu_ref, out_ref,
                  wd_s, wu_s, bd_s, *, bd_dim):
    n_e = wd_ref.shape[0]

    @pl.when(pl.program_id(0) == 0)
    def _build_weights():
        for e in range(n_e):
            sl = pl.ds(e * bd_dim, bd_dim)
            wd_s[:, sl] = wd_ref[e].astype(jnp.bfloat16)
            wu_s[sl, :] = wu_ref[e].astype(jnp.bfloat16)
            bd_s[0:1, sl] = bd_ref[pl.ds(e, 1), :]

    hb = h_ref[...]
    cid = jnp.reshape(cid_ref[...], (hb.shape[0], 1))
    h16 = hb.astype(jnp.bfloat16)
    z = jnp.dot(h16, wd_s[...], preferred_element_type=jnp.float32)
    z = z + bd_s[...]
    a = 0.5 * z * (1.0 + lax.erf(z * _INV_SQRT2))
    col_expert = lax.broadcasted_iota(jnp.int32, z.shape, 1) // bd_dim
    am = jnp.where(col_expert == cid, a, 0.0).astype(jnp.bfloat16)
    delta = jnp.dot(am, wu_s[...], preferred_element_type=jnp.float32)
    oh = (lax.broadcasted_iota(jnp.int32, (hb.shape[0], n_e), 1) == cid)
    bu_sel = jnp.dot(oh.astype(jnp.float32), bu_ref[...],
                     preferred_element_type=jnp.float32)
    out_ref[...] = hb + delta + bu_sel


def kernel(h, cluster_ids, W_down, b_down, W_up, b_up):
    B, D = h.shape
    E, _, BD = W_down.shape
    BT = 512

    return pl.pallas_call(
        functools.partial(_adapter_body, bd_dim=BD),
        grid=(B // BT,),
        in_specs=[
            pl.BlockSpec((BT,), lambda i: (i,)),
            pl.BlockSpec((BT, D), lambda i: (i, 0)),
            pl.BlockSpec((E, D, BD), lambda i: (0, 0, 0)),
            pl.BlockSpec((E, BD), lambda i: (0, 0)),
            pl.BlockSpec((E, BD, D), lambda i: (0, 0, 0)),
            pl.BlockSpec((E, D), lambda i: (0, 0)),
        ],
        out_specs=pl.BlockSpec((BT, D), lambda i: (i, 0)),
        out_shape=jax.ShapeDtypeStruct((B, D), jnp.float32),
        scratch_shapes=[
            pltpu.VMEM((D, E * BD), jnp.bfloat16),
            pltpu.VMEM((E * BD, D), jnp.bfloat16),
            pltpu.VMEM((1, E * BD), jnp.float32),
        ],
    )(cluster_ids, h, W_down, b_down, W_up, b_up)
